# baseline (speedup 1.0000x reference)
import numpy as np
import jax
import jax.numpy as jnp
from jax import lax
from jax.experimental import pallas as pl
from jax.experimental.pallas import tpu as pltpu
from jax.experimental.pallas import tpu_sc as plsc

N, E, D, DE = 10000, 320000, 128, 16
NC, NS = 2, 16
NW = NC * NS
EP = E // NW
C = 16
NCHUNK = EP // C
RPB = 624
FW = 2 * DE
RSD = float(1.0 / np.sqrt(D))

BN1 = 400


def _qkv_body(x_ref, wq, bq, wk, bk, wv, bv, we, qb_ref, kv_ref, qe_ref):
    xb = x_ref[...]
    q = (jnp.dot(xb, wq[...], preferred_element_type=jnp.float32) + bq[...]) * RSD
    qb_ref[...] = q.astype(jnp.bfloat16)
    qe_ref[...] = lax.dot_general(q, we[...], (((1,), (1,)), ((), ())),
                                  preferred_element_type=jnp.float32)
    k = jnp.dot(xb, wk[...], preferred_element_type=jnp.float32) + bk[...]
    v = jnp.dot(xb, wv[...], preferred_element_type=jnp.float32) + bv[...]
    kv_ref[:, :D] = k.astype(jnp.bfloat16)
    v4 = v.reshape(BN1, D // 32, 2, 16)
    vi = jnp.stack([v4[:, :, 0, :], v4[:, :, 1, :]], axis=-1).reshape(BN1, D)
    kv_ref[:, D:] = vi.astype(jnp.bfloat16)


def _qkv_call(x, Wq, bq, Wk, bk, Wv, bv, We):
    full = lambda shape: pl.BlockSpec(shape, lambda i: (0,) * len(shape))
    row = lambda w: pl.BlockSpec((BN1, w), lambda i: (i, 0))
    return pl.pallas_call(
        _qkv_body,
        grid=(N // BN1,),
        in_specs=[row(D), full((D, D)), full((1, D)), full((D, D)), full((1, D)),
                  full((D, D)), full((1, D)), full((DE, D))],
        out_specs=[row(D), row(2 * D), row(DE)],
        out_shape=[jax.ShapeDtypeStruct((N, D), jnp.bfloat16),
                   jax.ShapeDtypeStruct((N, 2 * D), jnp.bfloat16),
                   jax.ShapeDtypeStruct((N, DE), jnp.float32)],
    )(x, Wq, bq, Wk, bk, Wv, bv, We)


def _pack_body(s_ref, d_ref, o_ref):
    o_ref[...] = jnp.bitwise_or(jnp.left_shift(d_ref[...], 16), s_ref[...])


def _pack_call(src2, dst2):
    return pl.pallas_call(
        _pack_body,
        out_shape=jax.ShapeDtypeStruct(src2.shape, jnp.int32),
    )(src2, dst2)


def _edge_body(qb_hbm, kv_hbm, qe_hbm, ea_hbm, pk_hbm,
               zu_hbm, zf_hbm, u_out, f_out,
               pkbuf, qbb0, kvb0, qeb0, eab0, qbb1, kvb1, qeb1, eab1, vsc, psc,
               u_sh, f_sh, semg0, semg1):
    c = lax.axis_index("c")
    s = lax.axis_index("s")
    wid = s * NC + c
    r0 = pl.multiple_of(s * RPB, 16)

    pltpu.sync_copy(zu_hbm.at[pl.ds(r0, RPB), :], u_sh.at[pl.ds(r0, RPB), :])
    pltpu.sync_copy(zf_hbm.at[pl.ds(r0, RPB), :], f_sh.at[pl.ds(r0, RPB), :])

    @pl.when(s == NS - 1)
    def _init_tail():
        pltpu.sync_copy(zu_hbm.at[pl.ds(N - 16, 16), :], u_sh.at[pl.ds(N - 16, 16), :])
        pltpu.sync_copy(zf_hbm.at[pl.ds(N - 16, 16), :], f_sh.at[pl.ds(N - 16, 16), :])

    plsc.subcore_barrier()

    iot = lax.iota(jnp.int32, 16)
    lane0 = iot == 0
    zero16 = jnp.zeros((16,), jnp.float32)

    pltpu.sync_copy(pk_hbm.at[pl.ds(wid * EP, EP)], pkbuf)

    bufs = ((qbb0, kvb0, qeb0, eab0, semg0),
            (qbb1, kvb1, qeb1, eab1, semg1))

    def idx_of(j):
        pk16 = pkbuf[pl.ds(j * C, C)]
        return pk16 & 0xFFFF, lax.shift_right_logical(pk16, 16)

    def descs(j, p):
        srcv, dstv = idx_of(j)
        qbb, kvb, qeb, eab, sg = bufs[p]
        base = wid * EP + pl.multiple_of(j * C, C)
        return ((qb_hbm.at[dstv], qbb, sg),
                (kv_hbm.at[srcv], kvb, sg),
                (qe_hbm.at[dstv], qeb, sg),
                (ea_hbm.at[pl.ds(base, C), :], eab, sg))

    def issue(j, p):
        for d in descs(j, p):
            pltpu.async_copy(*d)

    def wait_for(j, p):
        for d in descs(j, p):
            pltpu.make_async_copy(*d).wait()

    def unpk(x32):
        return plsc.unpack(x32, format=plsc.PackFormat.INTERLEAVED,
                           preferred_element_type=jnp.float32)

    def compute(j, p):
        _, dstv = idx_of(j)
        qbb, kvb, qeb, eab, sg = bufs[p]
        def edot(l, av):
            acc = qeb[l, :] * eab[l, :]
            for db in range(D // 32):
                qa, qo = unpk(qbb[l, pl.ds(db * 32, 32)])
                ka, ko = unpk(kvb[l, pl.ds(db * 32, 32)])
                acc = acc + qa * ka + qo * ko
            return jnp.where(iot == l, jnp.full((16,), jnp.sum(acc), jnp.float32), av)

        av = lax.fori_loop(0, C, edot, zero16, unroll=4)
        ex16 = jnp.exp(av)

        for l in range(16):
            sv = jnp.full((16,), ex16[l], jnp.float32)
            for b in range(D // 32):
                lo, hi = unpk(kvb[l, pl.ds(D + b * 32, 32)])
                vsc[l, pl.ds(b * 32, 16)] = lo * sv
                vsc[l, pl.ds(b * 32 + 16, 16)] = hi * sv
            psc[l, pl.ds(0, 16)] = eab[l, :] * sv
            psc[l, pl.ds(16, 16)] = jnp.where(lane0, sv, zero16)
        pltpu.sync_copy(vsc, u_sh.at[dstv], add=True)
        pltpu.sync_copy(psc, f_sh.at[dstv], add=True)

    issue(0, 0)

    def pair(t, _):
        j0 = t * 2
        wait_for(j0, 0)
        issue(j0 + 1, 1)
        compute(j0, 0)
        wait_for(j0 + 1, 1)
        issue(j0 + 2, 0)
        compute(j0 + 1, 1)
        return 0

    lax.fori_loop(0, (NCHUNK - 1) // 2, pair, 0)
    wait_for(NCHUNK - 1, 0)
    compute(NCHUNK - 1, 0)

    plsc.subcore_barrier()

    pltpu.sync_copy(u_sh.at[pl.ds(r0, RPB), :], u_out.at[c, pl.ds(r0, RPB), :])
    pltpu.sync_copy(f_sh.at[pl.ds(r0, RPB), :], f_out.at[c, pl.ds(r0, RPB), :])

    @pl.when(s == NS - 1)
    def _out_tail():
        pltpu.sync_copy(u_sh.at[pl.ds(N - 16, 16), :], u_out.at[c, pl.ds(N - 16, 16), :])
        pltpu.sync_copy(f_sh.at[pl.ds(N - 16, 16), :], f_out.at[c, pl.ds(N - 16, 16), :])


_edge_pass = pl.kernel(
    _edge_body,
    out_type=[jax.ShapeDtypeStruct((NC, N, D), jnp.float32),
              jax.ShapeDtypeStruct((NC, N, FW), jnp.float32)],
    mesh=plsc.VectorSubcoreMesh(core_axis_name="c", subcore_axis_name="s"),
    compiler_params=pltpu.CompilerParams(needs_layout_passes=False,
                                         use_tc_tiling_on_sc=False),
    scratch_types=[
        pltpu.VMEM((EP,), jnp.int32),
        pltpu.VMEM((C, D), jnp.bfloat16),
        pltpu.VMEM((C, 2 * D), jnp.bfloat16),
        pltpu.VMEM((C, DE), jnp.float32),
        pltpu.VMEM((C, DE), jnp.float32),
        pltpu.VMEM((C, D), jnp.bfloat16),
        pltpu.VMEM((C, 2 * D), jnp.bfloat16),
        pltpu.VMEM((C, DE), jnp.float32),
        pltpu.VMEM((C, DE), jnp.float32),
        pltpu.VMEM((C, D), jnp.float32),
        pltpu.VMEM((C, FW), jnp.float32),
        pltpu.VMEM_SHARED((N, D), jnp.float32),
        pltpu.VMEM_SHARED((N, FW), jnp.float32),
        pltpu.SemaphoreType.DMA,
        pltpu.SemaphoreType.DMA,
    ],
)


BN2 = 1000


def _ln(x, g, b):
    mu = jnp.mean(x, axis=-1, keepdims=True)
    var = jnp.mean((x - mu) ** 2, axis=-1, keepdims=True)
    return (x - mu) / jnp.sqrt(var + 1e-5) * g + b


def _final_body(u_ref, f_ref, x_ref, we, wskip, bskip, g1, be1, g2, be2,
                w1, bf1, w2, bf2, o_ref):
    U = u_ref[0] + u_ref[1]
    Fp = f_ref[0] + f_ref[1]
    den = Fp[:, DE][:, None] + 1e-16
    agg = (U + jnp.dot(Fp[:, :DE], we[...], preferred_element_type=jnp.float32)) / den
    out = jnp.dot(agg, wskip[...], preferred_element_type=jnp.float32) + bskip[...]
    h = _ln(out + x_ref[...], g1[...], be1[...])
    ff = jnp.dot(
        jnp.maximum(jnp.dot(h, w1[...], preferred_element_type=jnp.float32) + bf1[...], 0.0),
        w2[...], preferred_element_type=jnp.float32) + bf2[...]
    o_ref[...] = _ln(h + ff, g2[...], be2[...])


def _final_call(u2, f2, x, We, Wskip, bskip, g1, be1, g2, be2, W1, bf1, W2, bf2):
    full = lambda shape: pl.BlockSpec(shape, lambda i: (0,) * len(shape))
    return pl.pallas_call(
        _final_body,
        grid=(N // BN2,),
        in_specs=[pl.BlockSpec((NC, BN2, D), lambda i: (0, i, 0)),
                  pl.BlockSpec((NC, BN2, FW), lambda i: (0, i, 0)),
                  pl.BlockSpec((BN2, D), lambda i: (i, 0)),
                  full((DE, D)), full((D, D)), full((1, D)), full((1, D)),
                  full((1, D)), full((1, D)), full((1, D)),
                  full((D, 2 * D)), full((1, 2 * D)), full((2 * D, D)), full((1, D))],
        out_specs=pl.BlockSpec((BN2, D), lambda i: (i, 0)),
        out_shape=jax.ShapeDtypeStruct((N, D), jnp.float32),
    )(u2, f2, x, We, Wskip, bskip, g1, be1, g2, be2, W1, bf1, W2, bf2)


def kernel(x, edge_index, edge_attr, Wq, bq, Wk, bk, Wv, bv, We, Wskip, bskip,
           g1, be1, g2, be2, W1, bf1, W2, bf2):
    qb, kv, qe = _qkv_call(x, Wq, bq.reshape(1, D), Wk, bk.reshape(1, D),
                           Wv, bv.reshape(1, D), We)
    pk = _pack_call(edge_index[0].reshape(E // 128, 128),
                    edge_index[1].reshape(E // 128, 128)).reshape(E)
    zu = jnp.zeros((N, D), jnp.float32)
    zf = jnp.zeros((N, FW), jnp.float32)
    u2, f2 = _edge_pass(qb, kv, qe, edge_attr, pk, zu, zf)
    return _final_call(u2, f2, x, We, Wskip, bskip.reshape(1, D),
                       g1.reshape(1, D), be1.reshape(1, D),
                       g2.reshape(1, D), be2.reshape(1, D),
                       W1, bf1.reshape(1, 2 * D), W2, bf2.reshape(1, D))


# speedup vs baseline: 1.3106x; 1.3106x over previous
import numpy as np
import jax
import jax.numpy as jnp
from jax import lax
from jax.experimental import pallas as pl
from jax.experimental.pallas import tpu as pltpu
from jax.experimental.pallas import tpu_sc as plsc

N, E, D, DE = 10000, 320000, 128, 16
NC, NS = 2, 16
NW = NC * NS
EP = E // NW
C = 16
NCHUNK = EP // C
RPB = 624
FW = 2 * DE
RSD = float(1.0 / np.sqrt(D))

_VPERM = np.arange(D).reshape(D // 32, 2, 16).transpose(0, 2, 1).reshape(D)

BN1 = 2000


def _qkv_body(x_ref, wq, bq, wk, bk, wv, bv, we, qb_ref, kv_ref, qe_ref):
    xb = x_ref[...]
    q = (jnp.dot(xb, wq[...], preferred_element_type=jnp.float32) + bq[...]) * RSD
    qb_ref[...] = q.astype(jnp.bfloat16)
    qe_ref[...] = lax.dot_general(q, we[...], (((1,), (1,)), ((), ())),
                                  preferred_element_type=jnp.float32)
    k = jnp.dot(xb, wk[...], preferred_element_type=jnp.float32) + bk[...]
    v = jnp.dot(xb, wv[...], preferred_element_type=jnp.float32) + bv[...]
    kv_ref[:, :D] = k.astype(jnp.bfloat16)
    kv_ref[:, D:] = v.astype(jnp.bfloat16)


def _qkv_call(x, Wq, bq, Wk, bk, Wv, bv, We):
    full = lambda shape: pl.BlockSpec(shape, lambda i: (0,) * len(shape))
    row = lambda w: pl.BlockSpec((BN1, w), lambda i: (i, 0))
    return pl.pallas_call(
        _qkv_body,
        grid=(N // BN1,),
        in_specs=[row(D), full((D, D)), full((1, D)), full((D, D)), full((1, D)),
                  full((D, D)), full((1, D)), full((DE, D))],
        out_specs=[row(D), row(2 * D), row(DE)],
        out_shape=[jax.ShapeDtypeStruct((N, D), jnp.bfloat16),
                   jax.ShapeDtypeStruct((N, 2 * D), jnp.bfloat16),
                   jax.ShapeDtypeStruct((N, DE), jnp.float32)],
    )(x, Wq, bq, Wk, bk, Wv, bv, We)


def _pack_body(s_ref, d_ref, o_ref):
    o_ref[...] = jnp.bitwise_or(jnp.left_shift(d_ref[...], 16), s_ref[...])


def _pack_call(src2, dst2):
    return pl.pallas_call(
        _pack_body,
        out_shape=jax.ShapeDtypeStruct(src2.shape, jnp.int32),
    )(src2, dst2)


def _edge_body(qb_hbm, kv_hbm, qe_hbm, ea_hbm, pk_hbm,
               zu_hbm, zf_hbm, u_out, f_out,
               pkbuf, qbb0, kvb0, qeb0, eab0, qbb1, kvb1, qeb1, eab1, vsc, psc,
               u_sh, f_sh, semg0, semg1):
    c = lax.axis_index("c")
    s = lax.axis_index("s")
    wid = s * NC + c
    r0 = pl.multiple_of(s * RPB, 16)

    pltpu.sync_copy(zu_hbm.at[pl.ds(r0, RPB), :], u_sh.at[pl.ds(r0, RPB), :])
    pltpu.sync_copy(zf_hbm.at[pl.ds(r0, RPB), :], f_sh.at[pl.ds(r0, RPB), :])

    @pl.when(s == NS - 1)
    def _init_tail():
        pltpu.sync_copy(zu_hbm.at[pl.ds(N - 16, 16), :], u_sh.at[pl.ds(N - 16, 16), :])
        pltpu.sync_copy(zf_hbm.at[pl.ds(N - 16, 16), :], f_sh.at[pl.ds(N - 16, 16), :])

    plsc.subcore_barrier()

    iot = lax.iota(jnp.int32, 16)
    lane0 = iot == 0
    zero16 = jnp.zeros((16,), jnp.float32)

    pltpu.sync_copy(pk_hbm.at[pl.ds(wid * EP, EP)], pkbuf)

    bufs = ((qbb0, kvb0, qeb0, eab0, semg0),
            (qbb1, kvb1, qeb1, eab1, semg1))

    def idx_of(j):
        pk16 = pkbuf[pl.ds(j * C, C)]
        return pk16 & 0xFFFF, lax.shift_right_logical(pk16, 16)

    def descs(j, p):
        srcv, dstv = idx_of(j)
        qbb, kvb, qeb, eab, sg = bufs[p]
        base = wid * EP + pl.multiple_of(j * C, C)
        return ((qb_hbm.at[dstv], qbb, sg),
                (kv_hbm.at[srcv], kvb, sg),
                (qe_hbm.at[dstv], qeb, sg),
                (ea_hbm.at[pl.ds(base, C), :], eab, sg))

    def issue(j, p):
        for d in descs(j, p):
            pltpu.async_copy(*d)

    def wait_for(j, p):
        for d in descs(j, p):
            pltpu.make_async_copy(*d).wait()

    def unpk(x32):
        return plsc.unpack(x32, format=plsc.PackFormat.INTERLEAVED,
                           preferred_element_type=jnp.float32)

    def compute(j, p):
        _, dstv = idx_of(j)
        qbb, kvb, qeb, eab, sg = bufs[p]
        def edot(l, av):
            acc = qeb[l, :] * eab[l, :]
            for db in range(D // 32):
                qa, qo = unpk(qbb[l, pl.ds(db * 32, 32)])
                ka, ko = unpk(kvb[l, pl.ds(db * 32, 32)])
                acc = acc + qa * ka + qo * ko
            return jnp.where(iot == l, jnp.full((16,), jnp.sum(acc), jnp.float32), av)

        av = lax.fori_loop(0, C, edot, zero16, unroll=4)
        ex16 = jnp.exp(av)

        for l in range(16):
            sv = jnp.full((16,), ex16[l], jnp.float32)
            for b in range(D // 32):
                lo, hi = unpk(kvb[l, pl.ds(D + b * 32, 32)])
                vsc[l, pl.ds(b * 32, 16)] = lo * sv
                vsc[l, pl.ds(b * 32 + 16, 16)] = hi * sv
            psc[l, pl.ds(0, 16)] = eab[l, :] * sv
            psc[l, pl.ds(16, 16)] = jnp.where(lane0, sv, zero16)
        pltpu.sync_copy(vsc, u_sh.at[dstv], add=True)
        pltpu.sync_copy(psc, f_sh.at[dstv], add=True)

    issue(0, 0)

    def pair(t, _):
        j0 = t * 2
        wait_for(j0, 0)
        issue(j0 + 1, 1)
        compute(j0, 0)
        wait_for(j0 + 1, 1)
        issue(j0 + 2, 0)
        compute(j0 + 1, 1)
        return 0

    lax.fori_loop(0, (NCHUNK - 1) // 2, pair, 0)
    wait_for(NCHUNK - 1, 0)
    compute(NCHUNK - 1, 0)

    plsc.subcore_barrier()

    pltpu.sync_copy(u_sh.at[pl.ds(r0, RPB), :], u_out.at[c, pl.ds(r0, RPB), :])
    pltpu.sync_copy(f_sh.at[pl.ds(r0, RPB), :], f_out.at[c, pl.ds(r0, RPB), :])

    @pl.when(s == NS - 1)
    def _out_tail():
        pltpu.sync_copy(u_sh.at[pl.ds(N - 16, 16), :], u_out.at[c, pl.ds(N - 16, 16), :])
        pltpu.sync_copy(f_sh.at[pl.ds(N - 16, 16), :], f_out.at[c, pl.ds(N - 16, 16), :])


_edge_pass = pl.kernel(
    _edge_body,
    out_type=[jax.ShapeDtypeStruct((NC, N, D), jnp.float32),
              jax.ShapeDtypeStruct((NC, N, FW), jnp.float32)],
    mesh=plsc.VectorSubcoreMesh(core_axis_name="c", subcore_axis_name="s"),
    compiler_params=pltpu.CompilerParams(needs_layout_passes=False,
                                         use_tc_tiling_on_sc=False),
    scratch_types=[
        pltpu.VMEM((EP,), jnp.int32),
        pltpu.VMEM((C, D), jnp.bfloat16),
        pltpu.VMEM((C, 2 * D), jnp.bfloat16),
        pltpu.VMEM((C, DE), jnp.float32),
        pltpu.VMEM((C, DE), jnp.float32),
        pltpu.VMEM((C, D), jnp.bfloat16),
        pltpu.VMEM((C, 2 * D), jnp.bfloat16),
        pltpu.VMEM((C, DE), jnp.float32),
        pltpu.VMEM((C, DE), jnp.float32),
        pltpu.VMEM((C, D), jnp.float32),
        pltpu.VMEM((C, FW), jnp.float32),
        pltpu.VMEM_SHARED((N, D), jnp.float32),
        pltpu.VMEM_SHARED((N, FW), jnp.float32),
        pltpu.SemaphoreType.DMA,
        pltpu.SemaphoreType.DMA,
    ],
)


BN2 = 1000


def _ln(x, g, b):
    mu = jnp.mean(x, axis=-1, keepdims=True)
    var = jnp.mean((x - mu) ** 2, axis=-1, keepdims=True)
    return (x - mu) / jnp.sqrt(var + 1e-5) * g + b


def _final_body(u_ref, f_ref, x_ref, we, wskip, bskip, g1, be1, g2, be2,
                w1, bf1, w2, bf2, o_ref):
    U = u_ref[0] + u_ref[1]
    Fp = f_ref[0] + f_ref[1]
    den = Fp[:, DE][:, None] + 1e-16
    agg = (U + jnp.dot(Fp[:, :DE], we[...], preferred_element_type=jnp.float32)) / den
    out = jnp.dot(agg, wskip[...], preferred_element_type=jnp.float32) + bskip[...]
    h = _ln(out + x_ref[...], g1[...], be1[...])
    ff = jnp.dot(
        jnp.maximum(jnp.dot(h, w1[...], preferred_element_type=jnp.float32) + bf1[...], 0.0),
        w2[...], preferred_element_type=jnp.float32) + bf2[...]
    o_ref[...] = _ln(h + ff, g2[...], be2[...])


def _final_call(u2, f2, x, We, Wskip, bskip, g1, be1, g2, be2, W1, bf1, W2, bf2):
    full = lambda shape: pl.BlockSpec(shape, lambda i: (0,) * len(shape))
    return pl.pallas_call(
        _final_body,
        grid=(N // BN2,),
        in_specs=[pl.BlockSpec((NC, BN2, D), lambda i: (0, i, 0)),
                  pl.BlockSpec((NC, BN2, FW), lambda i: (0, i, 0)),
                  pl.BlockSpec((BN2, D), lambda i: (i, 0)),
                  full((DE, D)), full((D, D)), full((1, D)), full((1, D)),
                  full((1, D)), full((1, D)), full((1, D)),
                  full((D, 2 * D)), full((1, 2 * D)), full((2 * D, D)), full((1, D))],
        out_specs=pl.BlockSpec((BN2, D), lambda i: (i, 0)),
        out_shape=jax.ShapeDtypeStruct((N, D), jnp.float32),
    )(u2, f2, x, We, Wskip, bskip, g1, be1, g2, be2, W1, bf1, W2, bf2)


def kernel(x, edge_index, edge_attr, Wq, bq, Wk, bk, Wv, bv, We, Wskip, bskip,
           g1, be1, g2, be2, W1, bf1, W2, bf2):
    qb, kv, qe = _qkv_call(x, Wq, bq.reshape(1, D), Wk, bk.reshape(1, D),
                           Wv[:, _VPERM], bv[_VPERM].reshape(1, D), We)
    pk = _pack_call(edge_index[0].reshape(E // 128, 128),
                    edge_index[1].reshape(E // 128, 128)).reshape(E)
    zu = jnp.zeros((N, D), jnp.float32)
    zf = jnp.zeros((N, FW), jnp.float32)
    u2, f2 = _edge_pass(qb, kv, qe, edge_attr, pk, zu, zf)
    return _final_call(u2, f2, x, We, Wskip, bskip.reshape(1, D),
                       g1.reshape(1, D), be1.reshape(1, D),
                       g2.reshape(1, D), be2.reshape(1, D),
                       W1, bf1.reshape(1, 2 * D), W2, bf2.reshape(1, D))


# speedup vs baseline: 1.7475x; 1.3333x over previous
import numpy as np
import jax
import jax.numpy as jnp
from jax import lax
from jax.experimental import pallas as pl
from jax.experimental.pallas import tpu as pltpu
from jax.experimental.pallas import tpu_sc as plsc

N, E, D, DE = 10000, 320000, 128, 16
NC, NS = 2, 16
NW = NC * NS
EP = E // NW
C = 16
NCHUNK = EP // C
RPB = 624
FW = 2 * DE
RSD = float(1.0 / np.sqrt(D))

_VPERM = np.arange(D).reshape(D // 32, 2, 16).transpose(0, 2, 1).reshape(D)

BN1 = 2000


def _qkv_body(x_ref, wq, bq, wk, bk, wv, bv, we, qb_ref, kv_ref, qe_ref):
    xb = x_ref[...]
    q = (jnp.dot(xb, wq[...], preferred_element_type=jnp.float32) + bq[...]) * RSD
    qb_ref[...] = q.astype(jnp.bfloat16)
    qe_ref[...] = lax.dot_general(q, we[...], (((1,), (1,)), ((), ())),
                                  preferred_element_type=jnp.float32)
    k = jnp.dot(xb, wk[...], preferred_element_type=jnp.float32) + bk[...]
    v = jnp.dot(xb, wv[...], preferred_element_type=jnp.float32) + bv[...]
    kv_ref[:, :D] = k.astype(jnp.bfloat16)
    kv_ref[:, D:] = v.astype(jnp.bfloat16)


def _qkv_call(x, Wq, bq, Wk, bk, Wv, bv, We):
    full = lambda shape: pl.BlockSpec(shape, lambda i: (0,) * len(shape))
    row = lambda w: pl.BlockSpec((BN1, w), lambda i: (i, 0))
    return pl.pallas_call(
        _qkv_body,
        grid=(N // BN1,),
        in_specs=[row(D), full((D, D)), full((1, D)), full((D, D)), full((1, D)),
                  full((D, D)), full((1, D)), full((DE, D))],
        out_specs=[row(D), row(2 * D), row(DE)],
        out_shape=[jax.ShapeDtypeStruct((N, D), jnp.bfloat16),
                   jax.ShapeDtypeStruct((N, 2 * D), jnp.bfloat16),
                   jax.ShapeDtypeStruct((N, DE), jnp.float32)],
    )(x, Wq, bq, Wk, bk, Wv, bv, We)


def _pack_body(s_ref, d_ref, o_ref):
    o_ref[...] = jnp.bitwise_or(jnp.left_shift(d_ref[...], 16), s_ref[...])


def _pack_call(src2, dst2):
    return pl.pallas_call(
        _pack_body,
        out_shape=jax.ShapeDtypeStruct(src2.shape, jnp.int32),
    )(src2, dst2)


def _edge_body(qb_hbm, kv_hbm, qe_hbm, ea_hbm, pk_hbm,
               zu_hbm, zf_hbm, u_out, f_out,
               pkbuf, qbb0, kvb0, qeb0, eab0, qbb1, kvb1, qeb1, eab1,
               qbb2, kvb2, qeb2, eab2, qbb3, kvb3, qeb3, eab3, vsc, psc,
               u_sh, f_sh, semg0, semg1, semg2, semg3):
    c = lax.axis_index("c")
    s = lax.axis_index("s")
    wid = s * NC + c
    r0 = pl.multiple_of(s * RPB, 16)

    pltpu.sync_copy(zu_hbm.at[pl.ds(r0, RPB), :], u_sh.at[pl.ds(r0, RPB), :])
    pltpu.sync_copy(zf_hbm.at[pl.ds(r0, RPB), :], f_sh.at[pl.ds(r0, RPB), :])

    @pl.when(s == NS - 1)
    def _init_tail():
        pltpu.sync_copy(zu_hbm.at[pl.ds(N - 16, 16), :], u_sh.at[pl.ds(N - 16, 16), :])
        pltpu.sync_copy(zf_hbm.at[pl.ds(N - 16, 16), :], f_sh.at[pl.ds(N - 16, 16), :])

    plsc.subcore_barrier()

    iot = lax.iota(jnp.int32, 16)
    lane0 = iot == 0
    zero16 = jnp.zeros((16,), jnp.float32)

    pltpu.sync_copy(pk_hbm.at[pl.ds(wid * EP, EP)], pkbuf)

    bufs = ((qbb0, kvb0, qeb0, eab0, semg0),
            (qbb1, kvb1, qeb1, eab1, semg1),
            (qbb2, kvb2, qeb2, eab2, semg2),
            (qbb3, kvb3, qeb3, eab3, semg3))

    def idx_of(j):
        pk16 = pkbuf[pl.ds(j * C, C)]
        return pk16 & 0xFFFF, lax.shift_right_logical(pk16, 16)

    def descs(j, p):
        srcv, dstv = idx_of(j)
        qbb, kvb, qeb, eab, sg = bufs[p]
        base = wid * EP + pl.multiple_of(j * C, C)
        return ((qb_hbm.at[dstv], qbb, sg),
                (kv_hbm.at[srcv], kvb, sg),
                (qe_hbm.at[dstv], qeb, sg),
                (ea_hbm.at[pl.ds(base, C), :], eab, sg))

    def issue(j, p):
        for d in descs(j, p):
            pltpu.async_copy(*d)

    def wait_for(j, p):
        for d in descs(j, p):
            pltpu.make_async_copy(*d).wait()

    def unpk(x32):
        return plsc.unpack(x32, format=plsc.PackFormat.INTERLEAVED,
                           preferred_element_type=jnp.float32)

    def compute(j, p):
        _, dstv = idx_of(j)
        qbb, kvb, qeb, eab, sg = bufs[p]
        def edot(l, av):
            acc = qeb[l, :] * eab[l, :]
            for db in range(D // 32):
                qa, qo = unpk(qbb[l, pl.ds(db * 32, 32)])
                ka, ko = unpk(kvb[l, pl.ds(db * 32, 32)])
                acc = acc + qa * ka + qo * ko
            return jnp.where(iot == l, jnp.full((16,), jnp.sum(acc), jnp.float32), av)

        av = lax.fori_loop(0, C, edot, zero16, unroll=4)
        ex16 = jnp.exp(av)

        for l in range(16):
            sv = jnp.full((16,), ex16[l], jnp.float32)
            for b in range(D // 32):
                lo, hi = unpk(kvb[l, pl.ds(D + b * 32, 32)])
                vsc[l, pl.ds(b * 32, 16)] = lo * sv
                vsc[l, pl.ds(b * 32 + 16, 16)] = hi * sv
            psc[l, pl.ds(0, 16)] = eab[l, :] * sv
            psc[l, pl.ds(16, 16)] = jnp.where(lane0, sv, zero16)
        pltpu.sync_copy(vsc, u_sh.at[dstv], add=True)
        pltpu.sync_copy(psc, f_sh.at[dstv], add=True)

    issue(0, 0)
    issue(1, 1)
    issue(2, 2)

    def quad(t, _):
        j0 = t * 4
        for b in range(4):
            j = j0 + b
            wait_for(j, b)

            @pl.when(j + 3 < NCHUNK)
            def _prefetch():
                issue(j + 3, (b + 3) % 4)

            compute(j, b)
        return 0

    lax.fori_loop(0, NCHUNK // 4, quad, 0)
    wait_for(NCHUNK - 1, 0)
    compute(NCHUNK - 1, 0)

    plsc.subcore_barrier()

    pltpu.sync_copy(u_sh.at[pl.ds(r0, RPB), :], u_out.at[c, pl.ds(r0, RPB), :])
    pltpu.sync_copy(f_sh.at[pl.ds(r0, RPB), :], f_out.at[c, pl.ds(r0, RPB), :])

    @pl.when(s == NS - 1)
    def _out_tail():
        pltpu.sync_copy(u_sh.at[pl.ds(N - 16, 16), :], u_out.at[c, pl.ds(N - 16, 16), :])
        pltpu.sync_copy(f_sh.at[pl.ds(N - 16, 16), :], f_out.at[c, pl.ds(N - 16, 16), :])


_edge_pass = pl.kernel(
    _edge_body,
    out_type=[jax.ShapeDtypeStruct((NC, N, D), jnp.float32),
              jax.ShapeDtypeStruct((NC, N, FW), jnp.float32)],
    mesh=plsc.VectorSubcoreMesh(core_axis_name="c", subcore_axis_name="s"),
    compiler_params=pltpu.CompilerParams(needs_layout_passes=False,
                                         use_tc_tiling_on_sc=False),
    scratch_types=[
        pltpu.VMEM((EP,), jnp.int32),
        pltpu.VMEM((C, D), jnp.bfloat16),
        pltpu.VMEM((C, 2 * D), jnp.bfloat16),
        pltpu.VMEM((C, DE), jnp.float32),
        pltpu.VMEM((C, DE), jnp.float32),
        pltpu.VMEM((C, D), jnp.bfloat16),
        pltpu.VMEM((C, 2 * D), jnp.bfloat16),
        pltpu.VMEM((C, DE), jnp.float32),
        pltpu.VMEM((C, DE), jnp.float32),
        pltpu.VMEM((C, D), jnp.bfloat16),
        pltpu.VMEM((C, 2 * D), jnp.bfloat16),
        pltpu.VMEM((C, DE), jnp.float32),
        pltpu.VMEM((C, DE), jnp.float32),
        pltpu.VMEM((C, D), jnp.bfloat16),
        pltpu.VMEM((C, 2 * D), jnp.bfloat16),
        pltpu.VMEM((C, DE), jnp.float32),
        pltpu.VMEM((C, DE), jnp.float32),
        pltpu.VMEM((C, D), jnp.float32),
        pltpu.VMEM((C, FW), jnp.float32),
        pltpu.VMEM_SHARED((N, D), jnp.float32),
        pltpu.VMEM_SHARED((N, FW), jnp.float32),
        pltpu.SemaphoreType.DMA,
        pltpu.SemaphoreType.DMA,
        pltpu.SemaphoreType.DMA,
        pltpu.SemaphoreType.DMA,
    ],
)


BN2 = 1000


def _ln(x, g, b):
    mu = jnp.mean(x, axis=-1, keepdims=True)
    var = jnp.mean((x - mu) ** 2, axis=-1, keepdims=True)
    return (x - mu) / jnp.sqrt(var + 1e-5) * g + b


def _final_body(u_ref, f_ref, x_ref, we, wskip, bskip, g1, be1, g2, be2,
                w1, bf1, w2, bf2, o_ref):
    U = u_ref[0] + u_ref[1]
    Fp = f_ref[0] + f_ref[1]
    den = Fp[:, DE][:, None] + 1e-16
    agg = (U + jnp.dot(Fp[:, :DE], we[...], preferred_element_type=jnp.float32)) / den
    out = jnp.dot(agg, wskip[...], preferred_element_type=jnp.float32) + bskip[...]
    h = _ln(out + x_ref[...], g1[...], be1[...])
    ff = jnp.dot(
        jnp.maximum(jnp.dot(h, w1[...], preferred_element_type=jnp.float32) + bf1[...], 0.0),
        w2[...], preferred_element_type=jnp.float32) + bf2[...]
    o_ref[...] = _ln(h + ff, g2[...], be2[...])


def _final_call(u2, f2, x, We, Wskip, bskip, g1, be1, g2, be2, W1, bf1, W2, bf2):
    full = lambda shape: pl.BlockSpec(shape, lambda i: (0,) * len(shape))
    return pl.pallas_call(
        _final_body,
        grid=(N // BN2,),
        in_specs=[pl.BlockSpec((NC, BN2, D), lambda i: (0, i, 0)),
                  pl.BlockSpec((NC, BN2, FW), lambda i: (0, i, 0)),
                  pl.BlockSpec((BN2, D), lambda i: (i, 0)),
                  full((DE, D)), full((D, D)), full((1, D)), full((1, D)),
                  full((1, D)), full((1, D)), full((1, D)),
                  full((D, 2 * D)), full((1, 2 * D)), full((2 * D, D)), full((1, D))],
        out_specs=pl.BlockSpec((BN2, D), lambda i: (i, 0)),
        out_shape=jax.ShapeDtypeStruct((N, D), jnp.float32),
    )(u2, f2, x, We, Wskip, bskip, g1, be1, g2, be2, W1, bf1, W2, bf2)


def kernel(x, edge_index, edge_attr, Wq, bq, Wk, bk, Wv, bv, We, Wskip, bskip,
           g1, be1, g2, be2, W1, bf1, W2, bf2):
    qb, kv, qe = _qkv_call(x, Wq, bq.reshape(1, D), Wk, bk.reshape(1, D),
                           Wv[:, _VPERM], bv[_VPERM].reshape(1, D), We)
    pk = _pack_call(edge_index[0].reshape(E // 128, 128),
                    edge_index[1].reshape(E // 128, 128)).reshape(E)
    zu = jnp.zeros((N, D), jnp.float32)
    zf = jnp.zeros((N, FW), jnp.float32)
    u2, f2 = _edge_pass(qb, kv, qe, edge_attr, pk, zu, zf)
    return _final_call(u2, f2, x, We, Wskip, bskip.reshape(1, D),
                       g1.reshape(1, D), be1.reshape(1, D),
                       g2.reshape(1, D), be2.reshape(1, D),
                       W1, bf1.reshape(1, 2 * D), W2, bf2.reshape(1, D))


# speedup vs baseline: 1.8484x; 1.0577x over previous
import numpy as np
import jax
import jax.numpy as jnp
from jax import lax
from jax.experimental import pallas as pl
from jax.experimental.pallas import tpu as pltpu
from jax.experimental.pallas import tpu_sc as plsc

N, E, D, DE = 10000, 320000, 128, 16
NC, NS = 2, 16
NW = NC * NS
EP = E // NW
C = 16
NCHUNK = EP // C
RPB = 624
PW = D + 2 * DE
RSD = float(1.0 / np.sqrt(D))

_VPERM = np.arange(D).reshape(D // 32, 2, 16).transpose(0, 2, 1).reshape(D)

BN1 = 2000


def _qkv_body(x_ref, wq, bq, wk, bk, wv, bv, we, s_ref, d_ref,
              qb_ref, kv_ref, qe_ref, pk_ref):
    @pl.when(pl.program_id(0) == 0)
    def _pack():
        pk_ref[...] = jnp.bitwise_or(jnp.left_shift(d_ref[...], 16), s_ref[...])

    xb = x_ref[...]
    q = (jnp.dot(xb, wq[...], preferred_element_type=jnp.float32) + bq[...]) * RSD
    qb_ref[...] = q.astype(jnp.bfloat16)
    qe_ref[...] = lax.dot_general(q, we[...], (((1,), (1,)), ((), ())),
                                  preferred_element_type=jnp.float32)
    k = jnp.dot(xb, wk[...], preferred_element_type=jnp.float32) + bk[...]
    v = jnp.dot(xb, wv[...], preferred_element_type=jnp.float32) + bv[...]
    kv_ref[:, :D] = k.astype(jnp.bfloat16)
    kv_ref[:, D:] = v.astype(jnp.bfloat16)


def _qkv_call(x, Wq, bq, Wk, bk, Wv, bv, We, src2, dst2):
    G = N // BN1
    full = lambda shape: pl.BlockSpec(shape, lambda i: (0,) * len(shape))
    row = lambda w: pl.BlockSpec((BN1, w), lambda i: (i, 0))
    erow = full((E // 128, 128))
    return pl.pallas_call(
        _qkv_body,
        grid=(G,),
        in_specs=[row(D), full((D, D)), full((1, D)), full((D, D)), full((1, D)),
                  full((D, D)), full((1, D)), full((DE, D)), erow, erow],
        out_specs=[row(D), row(2 * D), row(DE), erow],
        out_shape=[jax.ShapeDtypeStruct((N, D), jnp.bfloat16),
                   jax.ShapeDtypeStruct((N, 2 * D), jnp.bfloat16),
                   jax.ShapeDtypeStruct((N, DE), jnp.float32),
                   jax.ShapeDtypeStruct((E // 128, 128), jnp.int32)],
    )(x, Wq, bq, Wk, bk, Wv, bv, We, src2, dst2)


def _edge_body(qb_hbm, kv_hbm, qe_hbm, ea_hbm, pk_hbm,
               za_hbm, a_out,
               pkbuf, qbb0, kvb0, qeb0, eab0, qbb1, kvb1, qeb1, eab1,
               qbb2, kvb2, qeb2, eab2, qbb3, kvb3, qeb3, eab3, pbm,
               a_sh, semg0, semg1, semg2, semg3):
    c = lax.axis_index("c")
    s = lax.axis_index("s")
    wid = s * NC + c
    r0 = pl.multiple_of(s * RPB, 16)

    pltpu.sync_copy(za_hbm.at[pl.ds(r0, RPB), :], a_sh.at[pl.ds(r0, RPB), :])

    @pl.when(s == NS - 1)
    def _init_tail():
        pltpu.sync_copy(za_hbm.at[pl.ds(N - 16, 16), :], a_sh.at[pl.ds(N - 16, 16), :])

    plsc.subcore_barrier()

    iot = lax.iota(jnp.int32, 16)
    lane0 = iot == 0
    zero16 = jnp.zeros((16,), jnp.float32)

    pltpu.sync_copy(pk_hbm.at[pl.ds(wid * EP, EP)], pkbuf)

    bufs = ((qbb0, kvb0, qeb0, eab0, semg0),
            (qbb1, kvb1, qeb1, eab1, semg1),
            (qbb2, kvb2, qeb2, eab2, semg2),
            (qbb3, kvb3, qeb3, eab3, semg3))

    def idx_of(j):
        pk16 = pkbuf[pl.ds(j * C, C)]
        return pk16 & 0xFFFF, lax.shift_right_logical(pk16, 16)

    def descs(j, p):
        srcv, dstv = idx_of(j)
        qbb, kvb, qeb, eab, sg = bufs[p]
        base = wid * EP + pl.multiple_of(j * C, C)
        return ((qb_hbm.at[dstv], qbb, sg),
                (kv_hbm.at[srcv], kvb, sg),
                (qe_hbm.at[dstv], qeb, sg),
                (ea_hbm.at[pl.ds(base, C), :], eab, sg))

    def issue(j, p):
        for d in descs(j, p):
            pltpu.async_copy(*d)

    def wait_for(j, p):
        for d in descs(j, p):
            pltpu.make_async_copy(*d).wait()

    def unpk(x32):
        return plsc.unpack(x32, format=plsc.PackFormat.INTERLEAVED,
                           preferred_element_type=jnp.float32)

    def compute(j, p):
        _, dstv = idx_of(j)
        qbb, kvb, qeb, eab, sg = bufs[p]
        def edot(l, av):
            acc = qeb[l, :] * eab[l, :]
            for db in range(D // 32):
                qa, qo = unpk(qbb[l, pl.ds(db * 32, 32)])
                ka, ko = unpk(kvb[l, pl.ds(db * 32, 32)])
                acc = acc + qa * ka + qo * ko
            return jnp.where(iot == l, jnp.full((16,), jnp.sum(acc), jnp.float32), av)

        av = lax.fori_loop(0, C, edot, zero16, unroll=4)
        ex16 = jnp.exp(av)

        for l in range(16):
            sv = jnp.full((16,), ex16[l], jnp.float32)
            for b in range(D // 32):
                lo, hi = unpk(kvb[l, pl.ds(D + b * 32, 32)])
                pbm[l, pl.ds(b * 32, 16)] = lo * sv
                pbm[l, pl.ds(b * 32 + 16, 16)] = hi * sv
            pbm[l, pl.ds(D, 16)] = eab[l, :] * sv
            pbm[l, pl.ds(D + 16, 16)] = jnp.where(lane0, sv, zero16)
        pltpu.sync_copy(pbm, a_sh.at[dstv], add=True)

    issue(0, 0)
    issue(1, 1)
    issue(2, 2)

    def quad(t, _):
        j0 = t * 4
        for b in range(4):
            j = j0 + b
            wait_for(j, b)

            @pl.when(j + 3 < NCHUNK)
            def _prefetch():
                issue(j + 3, (b + 3) % 4)

            compute(j, b)
        return 0

    lax.fori_loop(0, NCHUNK // 4, quad, 0)
    wait_for(NCHUNK - 1, 0)
    compute(NCHUNK - 1, 0)

    plsc.subcore_barrier()

    pltpu.sync_copy(a_sh.at[pl.ds(r0, RPB), :], a_out.at[c, pl.ds(r0, RPB), :])

    @pl.when(s == NS - 1)
    def _out_tail():
        pltpu.sync_copy(a_sh.at[pl.ds(N - 16, 16), :], a_out.at[c, pl.ds(N - 16, 16), :])


_edge_pass = pl.kernel(
    _edge_body,
    out_type=jax.ShapeDtypeStruct((NC, N, PW), jnp.float32),
    mesh=plsc.VectorSubcoreMesh(core_axis_name="c", subcore_axis_name="s"),
    compiler_params=pltpu.CompilerParams(needs_layout_passes=False,
                                         use_tc_tiling_on_sc=False),
    scratch_types=[
        pltpu.VMEM((EP,), jnp.int32),
        pltpu.VMEM((C, D), jnp.bfloat16),
        pltpu.VMEM((C, 2 * D), jnp.bfloat16),
        pltpu.VMEM((C, DE), jnp.float32),
        pltpu.VMEM((C, DE), jnp.float32),
        pltpu.VMEM((C, D), jnp.bfloat16),
        pltpu.VMEM((C, 2 * D), jnp.bfloat16),
        pltpu.VMEM((C, DE), jnp.float32),
        pltpu.VMEM((C, DE), jnp.float32),
        pltpu.VMEM((C, D), jnp.bfloat16),
        pltpu.VMEM((C, 2 * D), jnp.bfloat16),
        pltpu.VMEM((C, DE), jnp.float32),
        pltpu.VMEM((C, DE), jnp.float32),
        pltpu.VMEM((C, D), jnp.bfloat16),
        pltpu.VMEM((C, 2 * D), jnp.bfloat16),
        pltpu.VMEM((C, DE), jnp.float32),
        pltpu.VMEM((C, DE), jnp.float32),
        pltpu.VMEM((C, PW), jnp.float32),
        pltpu.VMEM_SHARED((N, PW), jnp.float32),
        pltpu.SemaphoreType.DMA,
        pltpu.SemaphoreType.DMA,
        pltpu.SemaphoreType.DMA,
        pltpu.SemaphoreType.DMA,
    ],
)


BN2 = 1000


def _ln(x, g, b):
    mu = jnp.mean(x, axis=-1, keepdims=True)
    var = jnp.mean((x - mu) ** 2, axis=-1, keepdims=True)
    return (x - mu) / jnp.sqrt(var + 1e-5) * g + b


def _final_body(a_ref, x_ref, we, wskip, bskip, g1, be1, g2, be2,
                w1, bf1, w2, bf2, o_ref):
    A = a_ref[0] + a_ref[1]
    U = A[:, :D]
    den = A[:, D + DE][:, None] + 1e-16
    agg = (U + jnp.dot(A[:, D:D + DE], we[...], preferred_element_type=jnp.float32)) / den
    out = jnp.dot(agg, wskip[...], preferred_element_type=jnp.float32) + bskip[...]
    h = _ln(out + x_ref[...], g1[...], be1[...])
    ff = jnp.dot(
        jnp.maximum(jnp.dot(h, w1[...], preferred_element_type=jnp.float32) + bf1[...], 0.0),
        w2[...], preferred_element_type=jnp.float32) + bf2[...]
    o_ref[...] = _ln(h + ff, g2[...], be2[...])


def _final_call(a2, x, We, Wskip, bskip, g1, be1, g2, be2, W1, bf1, W2, bf2):
    full = lambda shape: pl.BlockSpec(shape, lambda i: (0,) * len(shape))
    return pl.pallas_call(
        _final_body,
        grid=(N // BN2,),
        in_specs=[pl.BlockSpec((NC, BN2, PW), lambda i: (0, i, 0)),
                  pl.BlockSpec((BN2, D), lambda i: (i, 0)),
                  full((DE, D)), full((D, D)), full((1, D)), full((1, D)),
                  full((1, D)), full((1, D)), full((1, D)),
                  full((D, 2 * D)), full((1, 2 * D)), full((2 * D, D)), full((1, D))],
        out_specs=pl.BlockSpec((BN2, D), lambda i: (i, 0)),
        out_shape=jax.ShapeDtypeStruct((N, D), jnp.float32),
    )(a2, x, We, Wskip, bskip, g1, be1, g2, be2, W1, bf1, W2, bf2)


def kernel(x, edge_index, edge_attr, Wq, bq, Wk, bk, Wv, bv, We, Wskip, bskip,
           g1, be1, g2, be2, W1, bf1, W2, bf2):
    qb, kv, qe, pk2 = _qkv_call(x, Wq, bq.reshape(1, D), Wk, bk.reshape(1, D),
                                Wv[:, _VPERM], bv[_VPERM].reshape(1, D), We,
                                edge_index[0].reshape(E // 128, 128),
                                edge_index[1].reshape(E // 128, 128))
    pk = pk2.reshape(E)
    za = jnp.zeros((N, PW), jnp.float32)
    a2 = _edge_pass(qb, kv, qe, edge_attr, pk, za)
    return _final_call(a2, x, We, Wskip, bskip.reshape(1, D),
                       g1.reshape(1, D), be1.reshape(1, D),
                       g2.reshape(1, D), be2.reshape(1, D),
                       W1, bf1.reshape(1, 2 * D), W2, bf2.reshape(1, D))


# speedup vs baseline: 2.1770x; 1.1778x over previous
import numpy as np
import jax
import jax.numpy as jnp
from jax import lax
from jax.experimental import pallas as pl
from jax.experimental.pallas import tpu as pltpu
from jax.experimental.pallas import tpu_sc as plsc

N, E, D, DE = 10000, 320000, 128, 16
NC, NS = 2, 16
NW = NC * NS
EP = E // NW
C = 16
NCHUNK = EP // C
RPB = 624
PW = D + 2 * DE
RSD = float(1.0 / np.sqrt(D))

_VPERM = np.arange(D).reshape(D // 32, 2, 16).transpose(0, 2, 1).reshape(D)

BN1 = 2000


def _qkv_body(x_ref, wq, bq, wk, bk, wv, bv, we, s_ref, d_ref,
              qb_ref, kv_ref, qe_ref, pk_ref):
    @pl.when(pl.program_id(0) == 0)
    def _pack():
        pk_ref[...] = jnp.bitwise_or(jnp.left_shift(d_ref[...], 16), s_ref[...])

    xb = x_ref[...]
    q = (jnp.dot(xb, wq[...], preferred_element_type=jnp.float32) + bq[...]) * RSD
    qb_ref[...] = q.astype(jnp.bfloat16)
    qe_ref[...] = lax.dot_general(q, we[...], (((1,), (1,)), ((), ())),
                                  preferred_element_type=jnp.float32)
    k = jnp.dot(xb, wk[...], preferred_element_type=jnp.float32) + bk[...]
    v = jnp.dot(xb, wv[...], preferred_element_type=jnp.float32) + bv[...]
    kv_ref[:, :D] = k.astype(jnp.bfloat16)
    kv_ref[:, D:] = v.astype(jnp.bfloat16)


def _qkv_call(x, Wq, bq, Wk, bk, Wv, bv, We, src2, dst2):
    G = N // BN1
    full = lambda shape: pl.BlockSpec(shape, lambda i: (0,) * len(shape))
    row = lambda w: pl.BlockSpec((BN1, w), lambda i: (i, 0))
    erow = full((E // 128, 128))
    return pl.pallas_call(
        _qkv_body,
        grid=(G,),
        in_specs=[row(D), full((D, D)), full((1, D)), full((D, D)), full((1, D)),
                  full((D, D)), full((1, D)), full((DE, D)), erow, erow],
        out_specs=[row(D), row(2 * D), row(DE), erow],
        out_shape=[jax.ShapeDtypeStruct((N, D), jnp.bfloat16),
                   jax.ShapeDtypeStruct((N, 2 * D), jnp.bfloat16),
                   jax.ShapeDtypeStruct((N, DE), jnp.float32),
                   jax.ShapeDtypeStruct((E // 128, 128), jnp.int32)],
    )(x, Wq, bq, Wk, bk, Wv, bv, We, src2, dst2)


def _edge_body(qb_hbm, kv_hbm, qe_hbm, ea_hbm, pk_hbm,
               za_hbm, a_out,
               pkbuf, qbb0, kvb0, qeb0, eab0, qbb1, kvb1, qeb1, eab1,
               qbb2, kvb2, qeb2, eab2, qbb3, kvb3, qeb3, eab3, pbm,
               a_sh, semg0, semg1, semg2, semg3, sems):
    c = lax.axis_index("c")
    s = lax.axis_index("s")
    wid = s * NC + c
    r0 = pl.multiple_of(s * RPB, 16)

    pltpu.sync_copy(za_hbm.at[pl.ds(r0, RPB), :], a_sh.at[pl.ds(r0, RPB), :])

    @pl.when(s == NS - 1)
    def _init_tail():
        pltpu.sync_copy(za_hbm.at[pl.ds(N - 16, 16), :], a_sh.at[pl.ds(N - 16, 16), :])

    plsc.subcore_barrier()

    iot = lax.iota(jnp.int32, 16)
    lane0 = iot == 0
    zero16 = jnp.zeros((16,), jnp.float32)

    pltpu.sync_copy(pk_hbm.at[pl.ds(wid * EP, EP)], pkbuf)

    bufs = ((qbb0, kvb0, qeb0, eab0, semg0),
            (qbb1, kvb1, qeb1, eab1, semg1),
            (qbb2, kvb2, qeb2, eab2, semg2),
            (qbb3, kvb3, qeb3, eab3, semg3))

    def idx_of(j):
        pk16 = pkbuf[pl.ds(j * C, C)]
        return pk16 & 0xFFFF, lax.shift_right_logical(pk16, 16)

    def descs(j, p):
        srcv, dstv = idx_of(j)
        qbb, kvb, qeb, eab, sg = bufs[p]
        base = wid * EP + pl.multiple_of(j * C, C)
        return ((qb_hbm.at[dstv], qbb, sg),
                (kv_hbm.at[srcv], kvb, sg),
                (qe_hbm.at[dstv], qeb, sg),
                (ea_hbm.at[pl.ds(base, C), :], eab, sg))

    def issue(j, p):
        for d in descs(j, p):
            pltpu.async_copy(*d)

    def wait_for(j, p):
        for d in descs(j, p):
            pltpu.make_async_copy(*d).wait()

    def unpk(x32):
        return plsc.unpack(x32, format=plsc.PackFormat.INTERLEAVED,
                           preferred_element_type=jnp.float32)

    def compute(j, p):
        _, dstv = idx_of(j)
        qbb, kvb, qeb, eab, sg = bufs[p]
        def edot(l, av):
            acc = qeb[l, :] * eab[l, :]
            for db in range(D // 32):
                qa, qo = unpk(qbb[l, pl.ds(db * 32, 32)])
                ka, ko = unpk(kvb[l, pl.ds(db * 32, 32)])
                acc = acc + qa * ka + qo * ko
            return jnp.where(iot == l, jnp.full((16,), jnp.sum(acc), jnp.float32), av)

        av = lax.fori_loop(0, C, edot, zero16, unroll=4)
        ex16 = jnp.exp(av)

        @pl.when(j >= 1)
        def _drain_prev():
            _, dprev = idx_of(j - 1)
            pltpu.make_async_copy(pbm, a_sh.at[dprev], sems).wait()

        for l in range(16):
            sv = jnp.full((16,), ex16[l], jnp.float32)
            for b in range(D // 32):
                lo, hi = unpk(kvb[l, pl.ds(D + b * 32, 32)])
                pbm[l, pl.ds(b * 32, 16)] = lo * sv
                pbm[l, pl.ds(b * 32 + 16, 16)] = hi * sv
            pbm[l, pl.ds(D, 16)] = eab[l, :] * sv
            pbm[l, pl.ds(D + 16, 16)] = jnp.where(lane0, sv, zero16)
        pltpu.async_copy(pbm, a_sh.at[dstv], sems, add=True)

    issue(0, 0)
    issue(1, 1)
    issue(2, 2)

    def quad(t, _):
        j0 = t * 4
        for b in range(4):
            j = j0 + b
            wait_for(j, b)

            @pl.when(j + 3 < NCHUNK)
            def _prefetch():
                issue(j + 3, (b + 3) % 4)

            compute(j, b)
        return 0

    lax.fori_loop(0, NCHUNK // 4, quad, 0)
    wait_for(NCHUNK - 1, 0)
    compute(NCHUNK - 1, 0)
    _, dlast = idx_of(NCHUNK - 1)
    pltpu.make_async_copy(pbm, a_sh.at[dlast], sems).wait()

    plsc.subcore_barrier()

    pltpu.sync_copy(a_sh.at[pl.ds(r0, RPB), :], a_out.at[c, pl.ds(r0, RPB), :])

    @pl.when(s == NS - 1)
    def _out_tail():
        pltpu.sync_copy(a_sh.at[pl.ds(N - 16, 16), :], a_out.at[c, pl.ds(N - 16, 16), :])


_edge_pass = pl.kernel(
    _edge_body,
    out_type=jax.ShapeDtypeStruct((NC, N, PW), jnp.float32),
    mesh=plsc.VectorSubcoreMesh(core_axis_name="c", subcore_axis_name="s"),
    compiler_params=pltpu.CompilerParams(needs_layout_passes=False,
                                         use_tc_tiling_on_sc=False),
    scratch_types=[
        pltpu.VMEM((EP,), jnp.int32),
        pltpu.VMEM((C, D), jnp.bfloat16),
        pltpu.VMEM((C, 2 * D), jnp.bfloat16),
        pltpu.VMEM((C, DE), jnp.float32),
        pltpu.VMEM((C, DE), jnp.float32),
        pltpu.VMEM((C, D), jnp.bfloat16),
        pltpu.VMEM((C, 2 * D), jnp.bfloat16),
        pltpu.VMEM((C, DE), jnp.float32),
        pltpu.VMEM((C, DE), jnp.float32),
        pltpu.VMEM((C, D), jnp.bfloat16),
        pltpu.VMEM((C, 2 * D), jnp.bfloat16),
        pltpu.VMEM((C, DE), jnp.float32),
        pltpu.VMEM((C, DE), jnp.float32),
        pltpu.VMEM((C, D), jnp.bfloat16),
        pltpu.VMEM((C, 2 * D), jnp.bfloat16),
        pltpu.VMEM((C, DE), jnp.float32),
        pltpu.VMEM((C, DE), jnp.float32),
        pltpu.VMEM((C, PW), jnp.float32),
        pltpu.VMEM_SHARED((N, PW), jnp.float32),
        pltpu.SemaphoreType.DMA,
        pltpu.SemaphoreType.DMA,
        pltpu.SemaphoreType.DMA,
        pltpu.SemaphoreType.DMA,
        pltpu.SemaphoreType.DMA,
    ],
)


BN2 = 1000


def _ln(x, g, b):
    mu = jnp.mean(x, axis=-1, keepdims=True)
    var = jnp.mean((x - mu) ** 2, axis=-1, keepdims=True)
    return (x - mu) / jnp.sqrt(var + 1e-5) * g + b


def _final_body(a_ref, x_ref, we, wskip, bskip, g1, be1, g2, be2,
                w1, bf1, w2, bf2, o_ref):
    A = a_ref[0] + a_ref[1]
    U = A[:, :D]
    den = A[:, D + DE][:, None] + 1e-16
    agg = (U + jnp.dot(A[:, D:D + DE], we[...], preferred_element_type=jnp.float32)) / den
    out = jnp.dot(agg, wskip[...], preferred_element_type=jnp.float32) + bskip[...]
    h = _ln(out + x_ref[...], g1[...], be1[...])
    ff = jnp.dot(
        jnp.maximum(jnp.dot(h, w1[...], preferred_element_type=jnp.float32) + bf1[...], 0.0),
        w2[...], preferred_element_type=jnp.float32) + bf2[...]
    o_ref[...] = _ln(h + ff, g2[...], be2[...])


def _final_call(a2, x, We, Wskip, bskip, g1, be1, g2, be2, W1, bf1, W2, bf2):
    full = lambda shape: pl.BlockSpec(shape, lambda i: (0,) * len(shape))
    return pl.pallas_call(
        _final_body,
        grid=(N // BN2,),
        in_specs=[pl.BlockSpec((NC, BN2, PW), lambda i: (0, i, 0)),
                  pl.BlockSpec((BN2, D), lambda i: (i, 0)),
                  full((DE, D)), full((D, D)), full((1, D)), full((1, D)),
                  full((1, D)), full((1, D)), full((1, D)),
                  full((D, 2 * D)), full((1, 2 * D)), full((2 * D, D)), full((1, D))],
        out_specs=pl.BlockSpec((BN2, D), lambda i: (i, 0)),
        out_shape=jax.ShapeDtypeStruct((N, D), jnp.float32),
    )(a2, x, We, Wskip, bskip, g1, be1, g2, be2, W1, bf1, W2, bf2)


def kernel(x, edge_index, edge_attr, Wq, bq, Wk, bk, Wv, bv, We, Wskip, bskip,
           g1, be1, g2, be2, W1, bf1, W2, bf2):
    qb, kv, qe, pk2 = _qkv_call(x, Wq, bq.reshape(1, D), Wk, bk.reshape(1, D),
                                Wv[:, _VPERM], bv[_VPERM].reshape(1, D), We,
                                edge_index[0].reshape(E // 128, 128),
                                edge_index[1].reshape(E // 128, 128))
    pk = pk2.reshape(E)
    za = jnp.zeros((N, PW), jnp.float32)
    a2 = _edge_pass(qb, kv, qe, edge_attr, pk, za)
    return _final_call(a2, x, We, Wskip, bskip.reshape(1, D),
                       g1.reshape(1, D), be1.reshape(1, D),
                       g2.reshape(1, D), be2.reshape(1, D),
                       W1, bf1.reshape(1, 2 * D), W2, bf2.reshape(1, D))
